# speedup vs baseline: 6.4257x; 6.4257x over previous
import functools

import jax
import jax.numpy as jnp
from jax import lax
from jax.experimental import pallas as pl
from jax.experimental.pallas import tpu as pltpu
from jax.experimental.pallas import tpu_sc as plsc

DIM = 128
NC = 2
NS = 16
NW = NC * NS
G = 128


def _emb_body(g_per_w, idx_hbm, tbl_hbm, out_hbm, idx_v, rows_v, sem):
    wid = lax.axis_index("s") * NC + lax.axis_index("c")
    gbase = wid * g_per_w
    rbase = gbase * G
    pltpu.sync_copy(idx_hbm.at[pl.ds(gbase, g_per_w)], idx_v)

    @pl.loop(0, g_per_w)
    def step(j):
        pltpu.async_copy(tbl_hbm.at[idx_v.at[j]], rows_v, sem).wait()
        pltpu.sync_copy(rows_v, out_hbm.at[pl.ds(rbase + j * G, G)])


def kernel(x, table):
    B, L = x.shape
    N = B * L
    assert N % (NW * G) == 0
    g_per_w = N // (NW * G)
    idx = x.reshape(N // G, G).astype(jnp.int32)

    mesh = plsc.VectorSubcoreMesh(core_axis_name="c", subcore_axis_name="s")
    emb = functools.partial(
        pl.kernel,
        out_type=jax.ShapeDtypeStruct((N, DIM), jnp.float32),
        mesh=mesh,
        scratch_types=[
            pltpu.VMEM((g_per_w, G), jnp.int32),
            pltpu.VMEM((G, DIM), jnp.float32),
            pltpu.SemaphoreType.DMA,
        ],
    )(functools.partial(_emb_body, g_per_w))

    out = emb(idx, table)
    return out.reshape(B, L, DIM)


# speedup vs baseline: 9.3610x; 1.4568x over previous
import functools

import jax
import jax.numpy as jnp
from jax import lax
from jax.experimental import pallas as pl
from jax.experimental.pallas import tpu as pltpu
from jax.experimental.pallas import tpu_sc as plsc

DIM = 128
NC = 2
NS = 16
NW = NC * NS
G = 128


def _emb_body(g_per_w, idx_hbm, tbl_hbm, out_hbm, idx_v, rows0, rows1,
              gsem0, gsem1, wsem0, wsem1):
    wid = lax.axis_index("s") * NC + lax.axis_index("c")
    gbase = wid * g_per_w
    rbase = gbase * G
    pltpu.sync_copy(idx_hbm.at[pl.ds(gbase, g_per_w)], idx_v)

    bufs = (rows0, rows1)
    gsems = (gsem0, gsem1)
    wsems = (wsem0, wsem1)

    def out_at(j):
        return out_hbm.at[pl.ds(rbase + j * G, G)]

    pltpu.async_copy(tbl_hbm.at[idx_v.at[0]], bufs[0], gsems[0])

    @pl.loop(0, g_per_w, step=2)
    def step(j0):
        for b in range(2):
            nb = 1 - b
            j = j0 + b

            @pl.when(j >= 1)
            def _():
                pltpu.make_async_copy(bufs[nb], out_at(j - 1), wsems[nb]).wait()

            @pl.when(j + 1 < g_per_w)
            def _():
                pltpu.async_copy(tbl_hbm.at[idx_v.at[j + 1]], bufs[nb], gsems[nb])

            pltpu.make_async_copy(tbl_hbm.at[idx_v.at[j]], bufs[b], gsems[b]).wait()
            pltpu.async_copy(bufs[b], out_at(j), wsems[b])

    lb = (g_per_w - 1) % 2
    pltpu.make_async_copy(bufs[lb], out_at(g_per_w - 1), wsems[lb]).wait()


def kernel(x, table):
    B, L = x.shape
    N = B * L
    assert N % (NW * G) == 0
    g_per_w = N // (NW * G)
    idx = x.reshape(N // G, G).astype(jnp.int32)

    mesh = plsc.VectorSubcoreMesh(core_axis_name="c", subcore_axis_name="s")
    emb = functools.partial(
        pl.kernel,
        out_type=jax.ShapeDtypeStruct((N, DIM), jnp.float32),
        mesh=mesh,
        scratch_types=[
            pltpu.VMEM((g_per_w, G), jnp.int32),
            pltpu.VMEM((G, DIM), jnp.float32),
            pltpu.VMEM((G, DIM), jnp.float32),
            pltpu.SemaphoreType.DMA,
            pltpu.SemaphoreType.DMA,
            pltpu.SemaphoreType.DMA,
            pltpu.SemaphoreType.DMA,
        ],
    )(functools.partial(_emb_body, g_per_w))

    out = emb(idx, table)
    return out.reshape(B, L, DIM)


# speedup vs baseline: 9.4079x; 1.0050x over previous
import functools

import jax
import jax.numpy as jnp
from jax import lax
from jax.experimental import pallas as pl
from jax.experimental.pallas import tpu as pltpu
from jax.experimental.pallas import tpu_sc as plsc

DIM = 128
NC = 2
NS = 16
NW = NC * NS
G = 128


NBUF = 4


def _emb_body(g_per_w, idx_hbm, tbl_hbm, out_hbm, idx_v, *rest):
    bufs = rest[:NBUF]
    gsems = rest[NBUF:2 * NBUF]
    wsems = rest[2 * NBUF:3 * NBUF]

    wid = lax.axis_index("s") * NC + lax.axis_index("c")
    gbase = wid * g_per_w
    rbase = gbase * G
    pltpu.sync_copy(idx_hbm.at[pl.ds(gbase, g_per_w)], idx_v)

    def out_at(j):
        return out_hbm.at[pl.ds(rbase + j * G, G)]

    for b in range(NBUF - 1):
        pltpu.async_copy(tbl_hbm.at[idx_v.at[b]], bufs[b], gsems[b])

    @pl.loop(0, g_per_w, step=NBUF)
    def step(j0):
        for b in range(NBUF):
            j = j0 + b
            nb = (b + NBUF - 1) % NBUF

            @pl.when(j >= 1)
            def _():
                pltpu.make_async_copy(bufs[nb], out_at(j - 1), wsems[nb]).wait()

            @pl.when(j + NBUF - 1 < g_per_w)
            def _():
                pltpu.async_copy(
                    tbl_hbm.at[idx_v.at[j + NBUF - 1]], bufs[nb], gsems[nb])

            pltpu.make_async_copy(tbl_hbm.at[idx_v.at[j]], bufs[b], gsems[b]).wait()
            pltpu.async_copy(bufs[b], out_at(j), wsems[b])

    lb = (g_per_w - 1) % NBUF
    pltpu.make_async_copy(bufs[lb], out_at(g_per_w - 1), wsems[lb]).wait()


def kernel(x, table):
    B, L = x.shape
    N = B * L
    assert N % (NW * G) == 0
    g_per_w = N // (NW * G)
    idx = x.reshape(N // G, G).astype(jnp.int32)

    mesh = plsc.VectorSubcoreMesh(core_axis_name="c", subcore_axis_name="s")
    emb = functools.partial(
        pl.kernel,
        out_type=jax.ShapeDtypeStruct((N, DIM), jnp.float32),
        mesh=mesh,
        scratch_types=(
            [pltpu.VMEM((g_per_w, G), jnp.int32)]
            + [pltpu.VMEM((G, DIM), jnp.float32) for _ in range(NBUF)]
            + [pltpu.SemaphoreType.DMA for _ in range(2 * NBUF)]
        ),
    )(functools.partial(_emb_body, g_per_w))

    out = emb(idx, table)
    return out.reshape(B, L, DIM)


# speedup vs baseline: 9.4270x; 1.0020x over previous
import functools

import jax
import jax.numpy as jnp
from jax import lax
from jax.experimental import pallas as pl
from jax.experimental.pallas import tpu as pltpu
from jax.experimental.pallas import tpu_sc as plsc

DIM = 128
NC = 2
NS = 16
NW = NC * NS
G = 128


NBUF = 5
AHEAD = 3


def _emb_body(g_per_w, idx_hbm, tbl_hbm, out_hbm, idx_v, *rest):
    bufs = rest[:NBUF]
    gsems = rest[NBUF:2 * NBUF]
    wsems = rest[2 * NBUF:3 * NBUF]
    slack = NBUF - AHEAD

    wid = lax.axis_index("s") * NC + lax.axis_index("c")
    gbase = wid * g_per_w
    rbase = gbase * G
    pltpu.sync_copy(idx_hbm.at[pl.ds(gbase, g_per_w)], idx_v)

    def out_at(j):
        return out_hbm.at[pl.ds(rbase + j * G, G)]

    for b in range(AHEAD):
        pltpu.async_copy(tbl_hbm.at[idx_v.at[b]], bufs[b], gsems[b])

    @pl.loop(0, g_per_w, step=NBUF)
    def step(j0):
        for b in range(NBUF):
            j = j0 + b
            nb = (b + AHEAD) % NBUF

            @pl.when(j >= slack)
            def _():
                pltpu.make_async_copy(bufs[nb], out_at(j - slack), wsems[nb]).wait()

            @pl.when(j + AHEAD < g_per_w)
            def _():
                pltpu.async_copy(
                    tbl_hbm.at[idx_v.at[j + AHEAD]], bufs[nb], gsems[nb])

            pltpu.make_async_copy(tbl_hbm.at[idx_v.at[j]], bufs[b], gsems[b]).wait()
            pltpu.async_copy(bufs[b], out_at(j), wsems[b])

    for t in range(slack):
        j = g_per_w - slack + t
        pltpu.make_async_copy(bufs[j % NBUF], out_at(j), wsems[j % NBUF]).wait()


def kernel(x, table):
    B, L = x.shape
    N = B * L
    assert N % (NW * G) == 0
    g_per_w = N // (NW * G)
    idx = x.reshape(N // G, G).astype(jnp.int32)

    mesh = plsc.VectorSubcoreMesh(core_axis_name="c", subcore_axis_name="s")
    emb = functools.partial(
        pl.kernel,
        out_type=jax.ShapeDtypeStruct((N, DIM), jnp.float32),
        mesh=mesh,
        scratch_types=(
            [pltpu.VMEM((g_per_w, G), jnp.int32)]
            + [pltpu.VMEM((G, DIM), jnp.float32) for _ in range(NBUF)]
            + [pltpu.SemaphoreType.DMA for _ in range(2 * NBUF)]
        ),
    )(functools.partial(_emb_body, g_per_w))

    out = emb(idx, table)
    return out.reshape(B, L, DIM)
